# baseline (speedup 1.0000x reference)
import jax
import jax.numpy as jnp
from jax import lax
from jax.experimental import pallas as pl

N = 10000
E = 320000
DIN = 128
DE = 16
DOUT = 128
H = 4
HD = 32


def _head_sum_matrix():
    r = lax.broadcasted_iota(jnp.int32, (DOUT, H), 0) // HD
    c = lax.broadcasted_iota(jnp.int32, (DOUT, H), 1)
    return (r == c).astype(jnp.float32)


def _k1_body(nf_ref, w1_ref, b_ref, af_ref, p_ref, sap_ref, gma_ref):
    i = pl.program_id(0)
    p = jnp.dot(nf_ref[...], w1_ref[...], preferred_element_type=jnp.float32)
    p = p + b_ref[...]
    p_ref[...] = p
    sap = jnp.dot(p * af_ref[...], _head_sum_matrix(),
                  preferred_element_type=jnp.float32)
    sap_ref[...] = sap
    bmax = jnp.max(sap, axis=0, keepdims=True)

    @pl.when(i == 0)
    def _():
        gma_ref[...] = bmax

    @pl.when(i > 0)
    def _():
        gma_ref[...] = jnp.maximum(gma_ref[...], bmax)


def _k2a_body(ef_ref, w2_ref, af_ref, sb_ref, gmb_ref):
    i = pl.program_id(0)
    v = jnp.dot(w2_ref[...] * af_ref[...], _head_sum_matrix(),
                preferred_element_type=jnp.float32)
    sb = jnp.dot(ef_ref[...], v, preferred_element_type=jnp.float32)
    sb_ref[...] = sb
    bmax = jnp.max(sb, axis=0, keepdims=True)

    @pl.when(i == 0)
    def _():
        gmb_ref[...] = bmax

    @pl.when(i > 0)
    def _():
        gmb_ref[...] = jnp.maximum(gmb_ref[...], bmax)


def _k2b_body(ef_ref, w2_ref, q_ref):
    q_ref[...] = jnp.dot(ef_ref[...], w2_ref[...],
                         preferred_element_type=jnp.float32)


def _k4_body(ss_ref, si_ref):
    sm = ss_ref[...]
    si_ref[...] = jnp.where(sm > 0.0, 1.0 / sm, 0.0)


def _kex_body(sa_ref, sb_ref, gb_ref, ex_ref):
    ex_ref[...] = jnp.exp(sa_ref[...] + sb_ref[...] - gb_ref[...])


def _kmsg_body(pg_ref, q_ref, ex_ref, si_ref, m_ref):
    r = lax.broadcasted_iota(jnp.int32, (H, DOUT), 0)
    cc = lax.broadcasted_iota(jnp.int32, (H, DOUT), 1) // HD
    gt = (r == cc).astype(jnp.float32)
    alpha = ex_ref[...] * si_ref[...]
    alpha_exp = jnp.dot(alpha, gt, preferred_element_type=jnp.float32)
    m_ref[...] = (pg_ref[...] + q_ref[...]) * alpha_exp


def _k6_body(acc_ref, nf_ref, w_ref, b_ref, g_ref, bb_ref, o_ref):
    hn = acc_ref[0] + acc_ref[1] + nf_ref[...]
    z = jnp.dot(hn, w_ref[...], preferred_element_type=jnp.float32)
    z = z + b_ref[...]
    h = jnp.maximum(z, 0.0)
    mu = jnp.mean(h, axis=-1, keepdims=True)
    vr = jnp.mean((h - mu) ** 2, axis=-1, keepdims=True)
    o_ref[...] = (h - mu) * lax.rsqrt(vr + 1e-5) * g_ref[...] + bb_ref[...]


def kernel(nfeats, efeats, edge_index, W_proj_w, W_proj_b, attn_vec,
           W_out_w, W_out_b, ln_gamma, ln_beta):
    f32 = jnp.float32
    src_idx = edge_index[0]
    dst_idx = edge_index[1]
    W1 = W_proj_w[:DIN]
    W2 = W_proj_w[DIN:]
    af = attn_vec.reshape(1, DOUT)
    bias = W_proj_b.reshape(1, DOUT)

    nblk = 2000
    k1 = pl.pallas_call(
        _k1_body,
        grid=(N // nblk,),
        in_specs=[
            pl.BlockSpec((nblk, DIN), lambda i: (i, 0)),
            pl.BlockSpec((DIN, DOUT), lambda i: (0, 0)),
            pl.BlockSpec((1, DOUT), lambda i: (0, 0)),
            pl.BlockSpec((1, DOUT), lambda i: (0, 0)),
        ],
        out_specs=[
            pl.BlockSpec((nblk, DOUT), lambda i: (i, 0)),
            pl.BlockSpec((nblk, H), lambda i: (i, 0)),
            pl.BlockSpec((1, H), lambda i: (0, 0)),
        ],
        out_shape=[
            jax.ShapeDtypeStruct((N, DOUT), f32),
            jax.ShapeDtypeStruct((N, H), f32),
            jax.ShapeDtypeStruct((1, H), f32),
        ],
    )
    P, sap, gma = k1(nfeats, W1, bias, af)

    eblk = 4000
    k2a = pl.pallas_call(
        _k2a_body,
        grid=(E // eblk,),
        in_specs=[
            pl.BlockSpec((eblk, DE), lambda i: (i, 0)),
            pl.BlockSpec((DE, DOUT), lambda i: (0, 0)),
            pl.BlockSpec((1, DOUT), lambda i: (0, 0)),
        ],
        out_specs=[
            pl.BlockSpec((eblk, H), lambda i: (i, 0)),
            pl.BlockSpec((1, H), lambda i: (0, 0)),
        ],
        out_shape=[
            jax.ShapeDtypeStruct((E, H), f32),
            jax.ShapeDtypeStruct((1, H), f32),
        ],
    )
    sb, gmb = k2a(efeats, W2, af)

    k2b = pl.pallas_call(
        _k2b_body,
        grid=(E // eblk,),
        in_specs=[
            pl.BlockSpec((eblk, DE), lambda i: (i, 0)),
            pl.BlockSpec((DE, DOUT), lambda i: (0, 0)),
        ],
        out_specs=pl.BlockSpec((eblk, DOUT), lambda i: (i, 0)),
        out_shape=jax.ShapeDtypeStruct((E, DOUT), f32),
    )
    q = k2b(efeats, W2)

    gb = (gma + gmb).reshape(1, H)

    sa_g = jnp.take(sap, src_idx, axis=0)
    kex = pl.pallas_call(
        _kex_body,
        grid=(E // eblk,),
        in_specs=[
            pl.BlockSpec((eblk, H), lambda i: (i, 0)),
            pl.BlockSpec((eblk, H), lambda i: (i, 0)),
            pl.BlockSpec((1, H), lambda i: (0, 0)),
        ],
        out_specs=pl.BlockSpec((eblk, H), lambda i: (i, 0)),
        out_shape=jax.ShapeDtypeStruct((E, H), f32),
    )
    ex = kex(sa_g, sb, gb)
    ssum = jax.ops.segment_sum(ex, dst_idx, num_segments=N)

    k4 = pl.pallas_call(
        _k4_body,
        grid=(5,),
        in_specs=[pl.BlockSpec((N // 5, H), lambda i: (i, 0))],
        out_specs=pl.BlockSpec((N // 5, H), lambda i: (i, 0)),
        out_shape=jax.ShapeDtypeStruct((N, H), f32),
    )
    sinv = k4(ssum)

    p_g = jnp.take(P, src_idx, axis=0)
    si_g = jnp.take(sinv, dst_idx, axis=0)
    kmsg = pl.pallas_call(
        _kmsg_body,
        grid=(E // eblk,),
        in_specs=[
            pl.BlockSpec((eblk, DOUT), lambda i: (i, 0)),
            pl.BlockSpec((eblk, DOUT), lambda i: (i, 0)),
            pl.BlockSpec((eblk, H), lambda i: (i, 0)),
            pl.BlockSpec((eblk, H), lambda i: (i, 0)),
        ],
        out_specs=pl.BlockSpec((eblk, DOUT), lambda i: (i, 0)),
        out_shape=jax.ShapeDtypeStruct((E, DOUT), f32),
    )
    msg = kmsg(p_g, q, ex, si_g)
    acc0 = jax.ops.segment_sum(msg, dst_idx, num_segments=N)
    acc = jnp.stack([acc0, jnp.zeros_like(acc0)])

    k6 = pl.pallas_call(
        _k6_body,
        grid=(N // nblk,),
        in_specs=[
            pl.BlockSpec((2, nblk, DOUT), lambda i: (0, i, 0)),
            pl.BlockSpec((nblk, DOUT), lambda i: (i, 0)),
            pl.BlockSpec((DOUT, DOUT), lambda i: (0, 0)),
            pl.BlockSpec((1, DOUT), lambda i: (0, 0)),
            pl.BlockSpec((1, DOUT), lambda i: (0, 0)),
            pl.BlockSpec((1, DOUT), lambda i: (0, 0)),
        ],
        out_specs=pl.BlockSpec((nblk, DOUT), lambda i: (i, 0)),
        out_shape=jax.ShapeDtypeStruct((N, DOUT), f32),
    )
    out = k6(acc, nfeats, W_out_w, W_out_b.reshape(1, DOUT),
             ln_gamma.reshape(1, DOUT), ln_beta.reshape(1, DOUT))
    return out
